# baseline (speedup 1.0000x reference)
import functools

import jax
import jax.numpy as jnp
from jax.experimental import pallas as pl
from jax.experimental.pallas import tpu as pltpu

H = 512
DK = 128
L_BLK = 4000


def _dnd_kernel(k_ref, keys_ref, vals_ref, out_ref, acc_ref, l_ref):
    i = pl.program_id(0)
    nsteps = pl.num_programs(0)

    @pl.when(i == 0)
    def _init():
        acc_ref[...] = jnp.zeros_like(acc_ref)
        l_ref[...] = jnp.zeros_like(l_ref)

    q = k_ref[...]
    qn = q / (jnp.sqrt(jnp.sum(q * q)) + 1e-8)
    q8 = jnp.broadcast_to(qn, (8, DK))

    keys = keys_ref[...]
    vals = vals_ref[...]

    s_raw = jax.lax.dot_general(q8, keys, (((1,), (1,)), ((), ())),
                                preferred_element_type=jnp.float32)
    ones8 = jnp.ones((8, DK), dtype=jnp.float32)
    n2 = jax.lax.dot_general(ones8, keys * keys, (((1,), (1,)), ((), ())),
                             preferred_element_type=jnp.float32)
    sims = s_raw / (jnp.sqrt(n2) + 1e-8)
    e = jnp.exp(sims)

    acc_ref[...] += jax.lax.dot_general(e, vals, (((1,), (0,)), ((), ())),
                                        preferred_element_type=jnp.float32)
    l_ref[...] += jnp.sum(e, axis=1, keepdims=True)

    @pl.when(i == nsteps - 1)
    def _finish():
        out_ref[...] = jnp.tanh(acc_ref[...] / l_ref[...])


def _dnd_retrieve(k_t, dnd_keys, dnd_vals):
    L = dnd_keys.shape[0]
    nsteps = L // L_BLK
    m8 = pl.pallas_call(
        _dnd_kernel,
        grid=(nsteps,),
        in_specs=[
            pl.BlockSpec((1, DK), lambda i: (0, 0)),
            pl.BlockSpec((L_BLK, DK), lambda i: (i, 0)),
            pl.BlockSpec((L_BLK, H), lambda i: (i, 0)),
        ],
        out_specs=pl.BlockSpec((8, H), lambda i: (0, 0)),
        out_shape=jax.ShapeDtypeStruct((8, H), jnp.float32),
        scratch_shapes=[
            pltpu.VMEM((8, H), jnp.float32),
            pltpu.VMEM((8, 1), jnp.float32),
        ],
    )(k_t, dnd_keys, dnd_vals)
    return m8


def _lstm_kernel(x_ref, h_ref, c_ref, wi_ref, wh_ref, b_ref, m_ref,
                 h_out, c_out, f_out, i_out, o_out, r_out):
    wx = jax.lax.dot_general(x_ref[...], wi_ref[...], (((1,), (1,)), ((), ())),
                             preferred_element_type=jnp.float32)
    wh = jax.lax.dot_general(h_ref[...], wh_ref[...], (((1,), (1,)), ((), ())),
                             preferred_element_type=jnp.float32)
    preact = wx + wh + b_ref[...]
    f_t = jax.nn.sigmoid(preact[:, :H])
    i_t = jax.nn.sigmoid(preact[:, H:2 * H])
    o_t = jax.nn.sigmoid(preact[:, 2 * H:3 * H])
    r_t = jax.nn.sigmoid(preact[:, 3 * H:4 * H])
    c_new = jnp.tanh(preact[:, 4 * H:])
    c_t = f_t * c_ref[...] + i_t * c_new
    m_t = m_ref[0:1, :]
    h_t = o_t * jnp.tanh(c_t) + r_t * m_t
    h_out[...] = h_t
    c_out[...] = c_t
    f_out[...] = f_t
    i_out[...] = i_t
    o_out[...] = o_t
    r_out[...] = r_t


def _lstm(x2, h2, c2, W_i2h, W_h2h, bias, m8):
    B, D_in = x2.shape
    BB = 128
    nb = B // BB
    G = W_i2h.shape[0]
    outs = pl.pallas_call(
        _lstm_kernel,
        grid=(nb,),
        in_specs=[
            pl.BlockSpec((BB, D_in), lambda i: (i, 0)),
            pl.BlockSpec((BB, H), lambda i: (i, 0)),
            pl.BlockSpec((BB, H), lambda i: (i, 0)),
            pl.BlockSpec((G, D_in), lambda i: (0, 0)),
            pl.BlockSpec((G, H), lambda i: (0, 0)),
            pl.BlockSpec((1, G), lambda i: (0, 0)),
            pl.BlockSpec((8, H), lambda i: (0, 0)),
        ],
        out_specs=[pl.BlockSpec((BB, H), lambda i: (i, 0))] * 6,
        out_shape=[jax.ShapeDtypeStruct((B, H), jnp.float32)] * 6,
    )(x2, h2, c2, W_i2h, W_h2h, bias, m8)
    return outs


@jax.jit
def kernel(x_t, h, c, k_t, W_i2h, b_i2h, W_h2h, b_h2h, dnd_keys, dnd_vals):
    B = x_t.shape[1]
    x2 = x_t.reshape(B, -1)
    h2 = h.reshape(B, -1)
    c2 = c.reshape(B, -1)
    kt = k_t.reshape(1, -1)
    bias = (b_i2h + b_h2h).reshape(1, -1)

    m8 = _dnd_retrieve(kt, dnd_keys, dnd_vals)
    h_t, c_t, f_t, i_t, o_t, r_t = _lstm(x2, h2, c2, W_i2h, W_h2h, bias, m8)

    h_t3 = h_t.reshape(1, B, -1)
    c_t3 = c_t.reshape(1, B, -1)
    return (h_t3, h_t3, c_t3, f_t, i_t, o_t, r_t)


# speedup vs baseline: 1.1186x; 1.1186x over previous
import functools

import jax
import jax.numpy as jnp
from jax.experimental import pallas as pl
from jax.experimental.pallas import tpu as pltpu

H = 512
DK = 128
L_BLK = 4000


def _dnd_kernel(k_ref, keys_ref, vals_ref, out_ref, acc_ref, l_ref):
    i = pl.program_id(0)
    nsteps = pl.num_programs(0)

    @pl.when(i == 0)
    def _init():
        acc_ref[...] = jnp.zeros_like(acc_ref)
        l_ref[...] = jnp.zeros_like(l_ref)

    q = k_ref[...]
    qn = q / (jnp.sqrt(jnp.sum(q * q)) + 1e-8)
    q8 = jnp.broadcast_to(qn, (8, DK))

    keys = keys_ref[...]
    vals = vals_ref[...]

    s_raw = jax.lax.dot_general(q8, keys, (((1,), (1,)), ((), ())),
                                preferred_element_type=jnp.float32)
    ones8 = jnp.ones((8, DK), dtype=jnp.float32)
    n2 = jax.lax.dot_general(ones8, keys * keys, (((1,), (1,)), ((), ())),
                             preferred_element_type=jnp.float32)
    sims = s_raw / (jnp.sqrt(n2) + 1e-8)
    e = jnp.exp(sims)

    acc_ref[...] += jax.lax.dot_general(e, vals, (((1,), (0,)), ((), ())),
                                        preferred_element_type=jnp.float32)
    l_ref[...] += jnp.sum(e, axis=1, keepdims=True)

    @pl.when(i == nsteps - 1)
    def _finish():
        out_ref[...] = jnp.tanh(acc_ref[...] / l_ref[...])


def _dnd_retrieve(k_t, dnd_keys, dnd_vals):
    L = dnd_keys.shape[0]
    nsteps = L // L_BLK
    m8 = pl.pallas_call(
        _dnd_kernel,
        grid=(nsteps,),
        in_specs=[
            pl.BlockSpec((1, DK), lambda i: (0, 0)),
            pl.BlockSpec((L_BLK, DK), lambda i: (i, 0)),
            pl.BlockSpec((L_BLK, H), lambda i: (i, 0)),
        ],
        out_specs=pl.BlockSpec((8, H), lambda i: (0, 0)),
        out_shape=jax.ShapeDtypeStruct((8, H), jnp.float32),
        scratch_shapes=[
            pltpu.VMEM((8, H), jnp.float32),
            pltpu.VMEM((8, 1), jnp.float32),
        ],
    )(k_t, dnd_keys, dnd_vals)
    return m8


def _lstm_kernel(x_ref, h_ref, c_ref, wi_ref, wh_ref, b_ref, m_ref,
                 h_out, c_out, f_out, i_out, o_out, r_out):
    wx = jax.lax.dot_general(x_ref[...], wi_ref[...], (((1,), (1,)), ((), ())),
                             preferred_element_type=jnp.float32)
    wh = jax.lax.dot_general(h_ref[...], wh_ref[...], (((1,), (1,)), ((), ())),
                             preferred_element_type=jnp.float32)
    preact = wx + wh + b_ref[...]
    f_t = jax.nn.sigmoid(preact[:, :H])
    i_t = jax.nn.sigmoid(preact[:, H:2 * H])
    o_t = jax.nn.sigmoid(preact[:, 2 * H:3 * H])
    r_t = jax.nn.sigmoid(preact[:, 3 * H:4 * H])
    c_new = jnp.tanh(preact[:, 4 * H:])
    c_t = f_t * c_ref[...] + i_t * c_new
    m_t = m_ref[0:1, :]
    h_t = o_t * jnp.tanh(c_t) + r_t * m_t
    h_out[...] = h_t
    c_out[...] = c_t
    f_out[...] = f_t
    i_out[...] = i_t
    o_out[...] = o_t
    r_out[...] = r_t


def _lstm(x2, h2, c2, W_i2h, W_h2h, bias, m8):
    B, D_in = x2.shape
    BB = 256
    nb = B // BB
    G = W_i2h.shape[0]
    outs = pl.pallas_call(
        _lstm_kernel,
        grid=(nb,),
        in_specs=[
            pl.BlockSpec((BB, D_in), lambda i: (i, 0)),
            pl.BlockSpec((BB, H), lambda i: (i, 0)),
            pl.BlockSpec((BB, H), lambda i: (i, 0)),
            pl.BlockSpec((G, D_in), lambda i: (0, 0)),
            pl.BlockSpec((G, H), lambda i: (0, 0)),
            pl.BlockSpec((1, G), lambda i: (0, 0)),
            pl.BlockSpec((8, H), lambda i: (0, 0)),
        ],
        out_specs=[pl.BlockSpec((BB, H), lambda i: (i, 0))] * 6,
        out_shape=[jax.ShapeDtypeStruct((B, H), jnp.float32)] * 6,
    )(x2, h2, c2, W_i2h, W_h2h, bias, m8)
    return outs


@jax.jit
def kernel(x_t, h, c, k_t, W_i2h, b_i2h, W_h2h, b_h2h, dnd_keys, dnd_vals):
    B = x_t.shape[1]
    x2 = x_t.reshape(B, -1)
    h2 = h.reshape(B, -1)
    c2 = c.reshape(B, -1)
    kt = k_t.reshape(1, -1)
    bias = (b_i2h + b_h2h).reshape(1, -1)

    m8 = _dnd_retrieve(kt, dnd_keys, dnd_vals)
    h_t, c_t, f_t, i_t, o_t, r_t = _lstm(x2, h2, c2, W_i2h, W_h2h, bias, m8)

    h_t3 = h_t.reshape(1, B, -1)
    c_t3 = c_t.reshape(1, B, -1)
    return (h_t3, h_t3, c_t3, f_t, i_t, o_t, r_t)
